# baseline (speedup 1.0000x reference)
import functools

import jax
import jax.numpy as jnp
from jax import lax
from jax.experimental import pallas as pl
from jax.experimental.pallas import tpu as pltpu
from jax.experimental.pallas import tpu_sc as plsc

BATCH = 16384
NUM_KCS = 1000
NUM_PROBLEMS = 1000000
NUM_STUDENTS = 100000
KMAIN = 999424
TAIL = NUM_PROBLEMS - KMAIN
KSTUD = 98304
THTAIL = NUM_STUDENTS - KSTUD
NUM_CORES = 2
NUM_SUBCORES = 16
NW = NUM_CORES * NUM_SUBCORES
BPW = BATCH // NW
L = 16
CHUNKS = BPW // L
EPSILON = 1e-8

_MESH = dict(core_axis_name="c", subcore_axis_name="s",
             num_cores=NUM_CORES, num_subcores=NUM_SUBCORES)
_PARAMS = dict(needs_layout_passes=False, use_tc_tiling_on_sc=False)


def _sigmoid(x):
    return 1.0 / (1.0 + jnp.exp(-x))


def _body_a(pT_hbm, pF_hbm, pG_hbm, pS_hbm, thm_hbm, tht_hbm,
            kc_hbm, pid_hbm, sid_hbm,
            pm_hbm, pk_out,
            pT_v, pF_v, pG_v, pS_v,
            kc_v, pid_v, sid_v, pm_v,
            ti0_v, ti1_v, ti2_v, ti3_v,
            th0_v, th1_v, th2_v, th3_v,
            tht_v, opk_v, sem, sem_idx, sem_rest):
    c = lax.axis_index("c")
    s = lax.axis_index("s")
    wid = s * NUM_CORES + c
    base = wid * BPW

    cp_pid = pltpu.async_copy(pid_hbm.at[pl.ds(base, BPW)], pid_v, sem_idx)
    cp_sid = pltpu.async_copy(sid_hbm.at[pl.ds(base, BPW)], sid_v, sem_idx)
    cp_kc = pltpu.async_copy(kc_hbm.at[pl.ds(base, BPW)], kc_v, sem_rest)
    cp_pT = pltpu.async_copy(pT_hbm, pT_v, sem_rest)
    cp_pF = pltpu.async_copy(pF_hbm, pF_v, sem_rest)
    cp_pG = pltpu.async_copy(pG_hbm, pG_v, sem_rest)
    cp_pS = pltpu.async_copy(pS_hbm, pS_v, sem_rest)
    cp_tt = pltpu.async_copy(tht_hbm, tht_v, sem_rest)
    cp_pid.wait()
    cp_sid.wait()

    def mkidx(i, carry):
        off = i * L
        pm_v[pl.ds(off, L)] = jnp.minimum(pid_v[pl.ds(off, L)], KMAIN - 1)
        sid = jnp.minimum(sid_v[pl.ds(off, L)], KSTUD - 1)
        phys = (lax.shift_right_logical(sid, 7) * 512
                + jnp.bitwise_and(sid, 127))
        ti0_v[pl.ds(off, L)] = phys
        ti1_v[pl.ds(off, L)] = phys + 128
        ti2_v[pl.ds(off, L)] = phys + 256
        ti3_v[pl.ds(off, L)] = phys + 384
        return carry

    lax.fori_loop(0, CHUNKS, mkidx, 0)
    cp_pm = pltpu.async_copy(pm_v, pm_hbm.at[pl.ds(base, BPW)], sem_idx)

    cp_t0 = pltpu.async_copy(thm_hbm.at[ti0_v], th0_v, sem)
    cp_t1 = pltpu.async_copy(thm_hbm.at[ti1_v], th1_v, sem)
    cp_t2 = pltpu.async_copy(thm_hbm.at[ti2_v], th2_v, sem)
    cp_t3 = pltpu.async_copy(thm_hbm.at[ti3_v], th3_v, sem)
    cp_kc.wait()
    cp_pT.wait()
    cp_pF.wait()
    cp_pG.wait()
    cp_pS.wait()

    def step_kc(i, carry):
        off = i * L
        kc = kc_v[pl.ds(off, L)]
        opk_v[pl.ds(off, L)] = plsc.load_gather(pT_v, [kc])
        opk_v[pl.ds(BPW + off, L)] = plsc.load_gather(pF_v, [kc])
        opk_v[pl.ds(2 * BPW + off, L)] = plsc.load_gather(pG_v, [kc])
        opk_v[pl.ds(3 * BPW + off, L)] = plsc.load_gather(pS_v, [kc])
        return carry

    lax.fori_loop(0, CHUNKS, step_kc, 0)
    cp_tt.wait()
    cp_t0.wait()
    cp_t1.wait()
    cp_t2.wait()
    cp_t3.wait()

    def step(i, carry):
        off = i * L
        sid = sid_v[pl.ds(off, L)]
        in_tail = sid >= KSTUD
        tix = jnp.maximum(sid - KSTUD, 0)
        th0 = jnp.where(in_tail, plsc.load_gather(tht_v, [tix]),
                        th0_v[pl.ds(off, L)])
        th1 = jnp.where(in_tail, plsc.load_gather(tht_v, [tix + THTAIL]),
                        th1_v[pl.ds(off, L)])
        th2 = jnp.where(in_tail, plsc.load_gather(tht_v, [tix + 2 * THTAIL]),
                        th2_v[pl.ds(off, L)])
        th3 = jnp.where(in_tail, plsc.load_gather(tht_v, [tix + 3 * THTAIL]),
                        th3_v[pl.ds(off, L)])
        opk_v[pl.ds(off, L)] = _sigmoid(opk_v[pl.ds(off, L)] + th0)
        opk_v[pl.ds(BPW + off, L)] = _sigmoid(
            opk_v[pl.ds(BPW + off, L)] - th1)
        opk_v[pl.ds(2 * BPW + off, L)] = opk_v[pl.ds(2 * BPW + off, L)] + th2
        opk_v[pl.ds(3 * BPW + off, L)] = opk_v[pl.ds(3 * BPW + off, L)] - th3
        return carry

    lax.fori_loop(0, CHUNKS, step, 0)

    pltpu.sync_copy(opk_v, pk_out.at[pl.ds(4 * base, 4 * BPW)])
    cp_pm.wait()


def _body_b(h_hbm, obs_hbm, om_hbm, sg_hbm, tails_hbm,
            pm_hbm, pid_hbm, pk_hbm,
            hnew_hbm, pcorr_hbm,
            pid_v, pm_v, om_v, sg_v, pk_v,
            tails_v, h_v, obs_v, hn_v, pc_v, sem, sem_idx, sem_rest):
    c = lax.axis_index("c")
    s = lax.axis_index("s")
    wid = s * NUM_CORES + c
    base = wid * BPW

    cp_pm = pltpu.async_copy(pm_hbm.at[pl.ds(base, BPW)], pm_v, sem_idx)
    cp_pid = pltpu.async_copy(pid_hbm.at[pl.ds(base, BPW)], pid_v, sem_rest)
    cp_h = pltpu.async_copy(h_hbm.at[pl.ds(2 * base, 2 * BPW)], h_v, sem_rest)
    cp_ob = pltpu.async_copy(obs_hbm.at[pl.ds(base, BPW)], obs_v, sem_rest)
    cp_pk = pltpu.async_copy(pk_hbm.at[pl.ds(4 * base, 4 * BPW)], pk_v,
                             sem_rest)
    cp_tl = pltpu.async_copy(tails_hbm, tails_v, sem_rest)
    cp_pm.wait()

    cp_om = pltpu.async_copy(om_hbm.at[pm_v], om_v, sem)
    cp_sg = pltpu.async_copy(sg_hbm.at[pm_v], sg_v, sem)
    cp_pid.wait()
    cp_h.wait()
    cp_ob.wait()
    cp_pk.wait()
    cp_tl.wait()
    cp_om.wait()
    cp_sg.wait()

    def step(i, carry):
        off = i * L
        pidc = pid_v[pl.ds(off, L)]
        in_tail = pidc >= KMAIN
        tidx = jnp.maximum(pidc - KMAIN, 0)
        om = jnp.where(in_tail, plsc.load_gather(tails_v, [tidx]),
                       om_v[pl.ds(off, L)])
        sg = jnp.where(in_tail, plsc.load_gather(tails_v, [tidx + TAIL]),
                       sg_v[pl.ds(off, L)])
        pT = pk_v[pl.ds(off, L)]
        pF = pk_v[pl.ds(BPW + off, L)]
        pG = _sigmoid(pk_v[pl.ds(2 * BPW + off, L)] + om)
        pS = _sigmoid(pk_v[pl.ds(3 * BPW + off, L)] + sg)
        hoff = (i >> 3) * 256 + (i & 7) * L
        h0 = h_v[pl.ds(hoff, L)]
        h1 = h_v[pl.ds(hoff + 128, L)]
        obs = obs_v[pl.ds(off, L)]

        obs_b = obs > 0.5
        p_m = jnp.where(obs_b, 1.0 - pS, pS)
        p_u = jnp.where(obs_b, pG, 1.0 - pG)
        a_u = p_u * h0
        a_m = p_m * h1
        nm = (1.0 - pF) * a_m + pT * a_u
        nu = pF * a_m + (1.0 - pT) * a_u
        inv = 1.0 / (nm + nu + EPSILON)
        nm = nm * inv
        nu = nu * inv
        pc = (1.0 - pS) * nm + pG * nu

        hn_v[pl.ds(hoff, L)] = nu
        hn_v[pl.ds(hoff + 128, L)] = nm
        pc_v[pl.ds(off, L)] = pc
        return carry

    lax.fori_loop(0, CHUNKS, step, 0)

    pltpu.sync_copy(hn_v, hnew_hbm.at[pl.ds(2 * base, 2 * BPW)])
    pltpu.sync_copy(pc_v, pcorr_hbm.at[pl.ds(base, BPW)])


@jax.jit
def _run(h_phys, observation, pT_logit, pF_logit, pG_logit, pS_logit,
         om_main, sg_main, tails, th_main, th_tail, kc_ids, pid, sid):
    fa = pl.kernel(
        _body_a,
        out_type=(
            jax.ShapeDtypeStruct((BATCH,), jnp.int32),
            jax.ShapeDtypeStruct((4 * BATCH,), jnp.float32),
        ),
        mesh=plsc.VectorSubcoreMesh(**_MESH),
        scratch_types=[
            pltpu.VMEM((NUM_KCS,), jnp.float32),
            pltpu.VMEM((NUM_KCS,), jnp.float32),
            pltpu.VMEM((NUM_KCS,), jnp.float32),
            pltpu.VMEM((NUM_KCS,), jnp.float32),
            pltpu.VMEM((BPW,), jnp.int32),
            pltpu.VMEM((BPW,), jnp.int32),
            pltpu.VMEM((BPW,), jnp.int32),
            pltpu.VMEM((BPW,), jnp.int32),
            pltpu.VMEM((BPW,), jnp.int32),
            pltpu.VMEM((BPW,), jnp.int32),
            pltpu.VMEM((BPW,), jnp.int32),
            pltpu.VMEM((BPW,), jnp.int32),
            pltpu.VMEM((BPW,), jnp.float32),
            pltpu.VMEM((BPW,), jnp.float32),
            pltpu.VMEM((BPW,), jnp.float32),
            pltpu.VMEM((BPW,), jnp.float32),
            pltpu.VMEM((4 * THTAIL,), jnp.float32),
            pltpu.VMEM((4 * BPW,), jnp.float32),
            pltpu.SemaphoreType.DMA,
            pltpu.SemaphoreType.DMA,
            pltpu.SemaphoreType.DMA,
        ],
        compiler_params=pltpu.CompilerParams(**_PARAMS),
        name="bkt_irt_a",
    )
    pm, pk = fa(pT_logit, pF_logit, pG_logit, pS_logit,
                th_main, th_tail, kc_ids, pid, sid)

    fb = pl.kernel(
        _body_b,
        out_type=(
            jax.ShapeDtypeStruct((2 * BATCH,), jnp.float32),
            jax.ShapeDtypeStruct((BATCH,), jnp.float32),
        ),
        mesh=plsc.VectorSubcoreMesh(**_MESH),
        scratch_types=[
            pltpu.VMEM((BPW,), jnp.int32),
            pltpu.VMEM((BPW,), jnp.int32),
            pltpu.VMEM((BPW,), jnp.float32),
            pltpu.VMEM((BPW,), jnp.float32),
            pltpu.VMEM((4 * BPW,), jnp.float32),
            pltpu.VMEM((2 * TAIL,), jnp.float32),
            pltpu.VMEM((2 * BPW,), jnp.float32),
            pltpu.VMEM((BPW,), jnp.float32),
            pltpu.VMEM((2 * BPW,), jnp.float32),
            pltpu.VMEM((BPW,), jnp.float32),
            pltpu.SemaphoreType.DMA,
            pltpu.SemaphoreType.DMA,
            pltpu.SemaphoreType.DMA,
        ],
        compiler_params=pltpu.CompilerParams(**_PARAMS),
        name="bkt_irt_b",
    )
    return fb(h_phys, observation, om_main, sg_main, tails,
              pm, pid, pk)


def kernel(h_prev, observation, pT_logit, pF_logit, pG_logit, pS_logit,
           omega_w, sigma_w, student_ability_w, kc_ids, problem_ids,
           student_ids):
    kc = kc_ids.astype(jnp.int32)
    pid = problem_ids.astype(jnp.int32)
    sid = student_ids.astype(jnp.int32)
    om_main = omega_w[:KMAIN].reshape(-1)
    sg_main = sigma_w[:KMAIN].reshape(-1)
    tails = jnp.concatenate([omega_w[KMAIN:], sigma_w[KMAIN:]],
                            axis=0).reshape(-1)
    th_main = (student_ability_w[:KSTUD]
               .reshape(KSTUD // 128, 128, 4).transpose(0, 2, 1).reshape(-1))
    th_tail = student_ability_w[KSTUD:].T.reshape(-1)
    h_phys = h_prev.reshape(128, 128, 2).transpose(0, 2, 1).reshape(-1)
    hn_flat, p_correct = _run(
        h_phys, observation, pT_logit, pF_logit, pG_logit, pS_logit,
        om_main, sg_main, tails, th_main, th_tail, kc, pid, sid)
    h_new = hn_flat.reshape(128, 2, 128).transpose(0, 2, 1).reshape(BATCH, 2)
    return (h_new, p_correct)


# speedup vs baseline: 1.0619x; 1.0619x over previous
import functools

import jax
import jax.numpy as jnp
from jax import lax
from jax.experimental import pallas as pl
from jax.experimental.pallas import tpu as pltpu
from jax.experimental.pallas import tpu_sc as plsc

BATCH = 16384
NUM_KCS = 1000
NUM_PROBLEMS = 1000000
NUM_STUDENTS = 100000
KMAIN = 999424
TAIL = NUM_PROBLEMS - KMAIN
NUM_CORES = 2
NUM_SUBCORES = 16
NW = NUM_CORES * NUM_SUBCORES
BPW = BATCH // NW
L = 16
CHUNKS = BPW // L
EPSILON = 1e-8

_MESH = dict(core_axis_name="c", subcore_axis_name="s",
             num_cores=NUM_CORES, num_subcores=NUM_SUBCORES)
_PARAMS = dict(needs_layout_passes=False, use_tc_tiling_on_sc=False)


def _sigmoid(x):
    return 1.0 / (1.0 + jnp.exp(-x))


def _body_a(pT_hbm, pF_hbm, pG_hbm, pS_hbm, th_hbm, kc_hbm, pid_hbm, sid_hbm,
            pm_hbm, pk_out,
            pT_v, pF_v, pG_v, pS_v,
            kc_v, pid_v, sid_v, pm_v,
            ti0_v, ti1_v, ti2_v, ti3_v,
            th0_v, th1_v, th2_v, th3_v,
            opk_v, sem, sem_idx, sem_rest):
    c = lax.axis_index("c")
    s = lax.axis_index("s")
    wid = s * NUM_CORES + c
    base = wid * BPW

    cp_pid = pltpu.async_copy(pid_hbm.at[pl.ds(base, BPW)], pid_v, sem_idx)
    cp_sid = pltpu.async_copy(sid_hbm.at[pl.ds(base, BPW)], sid_v, sem_idx)
    cp_kc = pltpu.async_copy(kc_hbm.at[pl.ds(base, BPW)], kc_v, sem_rest)
    cp_pT = pltpu.async_copy(pT_hbm, pT_v, sem_rest)
    cp_pF = pltpu.async_copy(pF_hbm, pF_v, sem_rest)
    cp_pG = pltpu.async_copy(pG_hbm, pG_v, sem_rest)
    cp_pS = pltpu.async_copy(pS_hbm, pS_v, sem_rest)
    cp_pid.wait()
    cp_sid.wait()

    def mkidx(i, carry):
        off = i * L
        pm_v[pl.ds(off, L)] = jnp.minimum(pid_v[pl.ds(off, L)], KMAIN - 1)
        sid = sid_v[pl.ds(off, L)]
        ti0_v[pl.ds(off, L)] = sid
        ti1_v[pl.ds(off, L)] = sid + NUM_STUDENTS
        ti2_v[pl.ds(off, L)] = sid + 2 * NUM_STUDENTS
        ti3_v[pl.ds(off, L)] = sid + 3 * NUM_STUDENTS
        return carry

    lax.fori_loop(0, CHUNKS, mkidx, 0)
    cp_pm = pltpu.async_copy(pm_v, pm_hbm.at[pl.ds(base, BPW)], sem_idx)

    cp_t0 = pltpu.async_copy(th_hbm.at[ti0_v], th0_v, sem)
    cp_t1 = pltpu.async_copy(th_hbm.at[ti1_v], th1_v, sem)
    cp_t2 = pltpu.async_copy(th_hbm.at[ti2_v], th2_v, sem)
    cp_t3 = pltpu.async_copy(th_hbm.at[ti3_v], th3_v, sem)
    cp_kc.wait()
    cp_pT.wait()
    cp_pF.wait()
    cp_pG.wait()
    cp_pS.wait()

    def step_kc(i, carry):
        off = i * L
        kc = kc_v[pl.ds(off, L)]
        opk_v[pl.ds(off, L)] = plsc.load_gather(pT_v, [kc])
        opk_v[pl.ds(BPW + off, L)] = plsc.load_gather(pF_v, [kc])
        opk_v[pl.ds(2 * BPW + off, L)] = plsc.load_gather(pG_v, [kc])
        opk_v[pl.ds(3 * BPW + off, L)] = plsc.load_gather(pS_v, [kc])
        return carry

    lax.fori_loop(0, CHUNKS, step_kc, 0)
    cp_t0.wait()
    cp_t1.wait()
    cp_t2.wait()
    cp_t3.wait()

    def step(i, carry):
        off = i * L
        opk_v[pl.ds(off, L)] = _sigmoid(
            opk_v[pl.ds(off, L)] + th0_v[pl.ds(off, L)])
        opk_v[pl.ds(BPW + off, L)] = _sigmoid(
            opk_v[pl.ds(BPW + off, L)] - th1_v[pl.ds(off, L)])
        opk_v[pl.ds(2 * BPW + off, L)] = (
            opk_v[pl.ds(2 * BPW + off, L)] + th2_v[pl.ds(off, L)])
        opk_v[pl.ds(3 * BPW + off, L)] = (
            opk_v[pl.ds(3 * BPW + off, L)] - th3_v[pl.ds(off, L)])
        return carry

    lax.fori_loop(0, CHUNKS, step, 0)

    pltpu.sync_copy(opk_v, pk_out.at[pl.ds(4 * base, 4 * BPW)])
    cp_pm.wait()


def _body_b(h_hbm, obs_hbm, om_hbm, sg_hbm, tails_hbm,
            pm_hbm, pid_hbm, pk_hbm,
            hnew_hbm, pcorr_hbm,
            pid_v, pm_v, om_v, sg_v, pk_v,
            tails_v, h_v, obs_v, hn_v, pc_v, sem, sem_idx, sem_rest):
    c = lax.axis_index("c")
    s = lax.axis_index("s")
    wid = s * NUM_CORES + c
    base = wid * BPW

    cp_pm = pltpu.async_copy(pm_hbm.at[pl.ds(base, BPW)], pm_v, sem_idx)
    cp_pid = pltpu.async_copy(pid_hbm.at[pl.ds(base, BPW)], pid_v, sem_rest)
    cp_h = pltpu.async_copy(h_hbm.at[pl.ds(2 * base, 2 * BPW)], h_v, sem_rest)
    cp_ob = pltpu.async_copy(obs_hbm.at[pl.ds(base, BPW)], obs_v, sem_rest)
    cp_pk = pltpu.async_copy(pk_hbm.at[pl.ds(4 * base, 4 * BPW)], pk_v,
                             sem_rest)
    cp_tl = pltpu.async_copy(tails_hbm, tails_v, sem_rest)
    cp_pm.wait()

    cp_om = pltpu.async_copy(om_hbm.at[pm_v], om_v, sem)
    cp_sg = pltpu.async_copy(sg_hbm.at[pm_v], sg_v, sem)
    cp_pid.wait()
    cp_h.wait()
    cp_ob.wait()
    cp_pk.wait()
    cp_tl.wait()
    cp_om.wait()
    cp_sg.wait()

    def step(i, carry):
        off = i * L
        pidc = pid_v[pl.ds(off, L)]
        in_tail = pidc >= KMAIN
        tidx = jnp.maximum(pidc - KMAIN, 0)
        om = jnp.where(in_tail, plsc.load_gather(tails_v, [tidx]),
                       om_v[pl.ds(off, L)])
        sg = jnp.where(in_tail, plsc.load_gather(tails_v, [tidx + TAIL]),
                       sg_v[pl.ds(off, L)])
        pT = pk_v[pl.ds(off, L)]
        pF = pk_v[pl.ds(BPW + off, L)]
        pG = _sigmoid(pk_v[pl.ds(2 * BPW + off, L)] + om)
        pS = _sigmoid(pk_v[pl.ds(3 * BPW + off, L)] + sg)
        hoff = (i >> 3) * 256 + (i & 7) * L
        h0 = h_v[pl.ds(hoff, L)]
        h1 = h_v[pl.ds(hoff + 128, L)]
        obs = obs_v[pl.ds(off, L)]

        obs_b = obs > 0.5
        p_m = jnp.where(obs_b, 1.0 - pS, pS)
        p_u = jnp.where(obs_b, pG, 1.0 - pG)
        a_u = p_u * h0
        a_m = p_m * h1
        nm = (1.0 - pF) * a_m + pT * a_u
        nu = pF * a_m + (1.0 - pT) * a_u
        inv = 1.0 / (nm + nu + EPSILON)
        nm = nm * inv
        nu = nu * inv
        pc = (1.0 - pS) * nm + pG * nu

        hn_v[pl.ds(hoff, L)] = nu
        hn_v[pl.ds(hoff + 128, L)] = nm
        pc_v[pl.ds(off, L)] = pc
        return carry

    lax.fori_loop(0, CHUNKS, step, 0)

    pltpu.sync_copy(hn_v, hnew_hbm.at[pl.ds(2 * base, 2 * BPW)])
    pltpu.sync_copy(pc_v, pcorr_hbm.at[pl.ds(base, BPW)])


@jax.jit
def _run(h_phys, observation, pT_logit, pF_logit, pG_logit, pS_logit,
         om_main, sg_main, tails, theta_flat, kc_ids, pid, sid):
    fa = pl.kernel(
        _body_a,
        out_type=(
            jax.ShapeDtypeStruct((BATCH,), jnp.int32),
            jax.ShapeDtypeStruct((4 * BATCH,), jnp.float32),
        ),
        mesh=plsc.VectorSubcoreMesh(**_MESH),
        scratch_types=[
            pltpu.VMEM((NUM_KCS,), jnp.float32),
            pltpu.VMEM((NUM_KCS,), jnp.float32),
            pltpu.VMEM((NUM_KCS,), jnp.float32),
            pltpu.VMEM((NUM_KCS,), jnp.float32),
            pltpu.VMEM((BPW,), jnp.int32),
            pltpu.VMEM((BPW,), jnp.int32),
            pltpu.VMEM((BPW,), jnp.int32),
            pltpu.VMEM((BPW,), jnp.int32),
            pltpu.VMEM((BPW,), jnp.int32),
            pltpu.VMEM((BPW,), jnp.int32),
            pltpu.VMEM((BPW,), jnp.int32),
            pltpu.VMEM((BPW,), jnp.int32),
            pltpu.VMEM((BPW,), jnp.float32),
            pltpu.VMEM((BPW,), jnp.float32),
            pltpu.VMEM((BPW,), jnp.float32),
            pltpu.VMEM((BPW,), jnp.float32),
            pltpu.VMEM((4 * BPW,), jnp.float32),
            pltpu.SemaphoreType.DMA,
            pltpu.SemaphoreType.DMA,
            pltpu.SemaphoreType.DMA,
        ],
        compiler_params=pltpu.CompilerParams(**_PARAMS),
        name="bkt_irt_a",
    )
    pm, pk = fa(pT_logit, pF_logit, pG_logit, pS_logit,
                theta_flat, kc_ids, pid, sid)

    fb = pl.kernel(
        _body_b,
        out_type=(
            jax.ShapeDtypeStruct((2 * BATCH,), jnp.float32),
            jax.ShapeDtypeStruct((BATCH,), jnp.float32),
        ),
        mesh=plsc.VectorSubcoreMesh(**_MESH),
        scratch_types=[
            pltpu.VMEM((BPW,), jnp.int32),
            pltpu.VMEM((BPW,), jnp.int32),
            pltpu.VMEM((BPW,), jnp.float32),
            pltpu.VMEM((BPW,), jnp.float32),
            pltpu.VMEM((4 * BPW,), jnp.float32),
            pltpu.VMEM((2 * TAIL,), jnp.float32),
            pltpu.VMEM((2 * BPW,), jnp.float32),
            pltpu.VMEM((BPW,), jnp.float32),
            pltpu.VMEM((2 * BPW,), jnp.float32),
            pltpu.VMEM((BPW,), jnp.float32),
            pltpu.SemaphoreType.DMA,
            pltpu.SemaphoreType.DMA,
            pltpu.SemaphoreType.DMA,
        ],
        compiler_params=pltpu.CompilerParams(**_PARAMS),
        name="bkt_irt_b",
    )
    return fb(h_phys, observation, om_main, sg_main, tails,
              pm, pid, pk)


def kernel(h_prev, observation, pT_logit, pF_logit, pG_logit, pS_logit,
           omega_w, sigma_w, student_ability_w, kc_ids, problem_ids,
           student_ids):
    kc = kc_ids.astype(jnp.int32)
    pid = problem_ids.astype(jnp.int32)
    sid = student_ids.astype(jnp.int32)
    om_main = omega_w[:KMAIN].reshape(-1)
    sg_main = sigma_w[:KMAIN].reshape(-1)
    tails = jnp.concatenate([omega_w[KMAIN:], sigma_w[KMAIN:]],
                            axis=0).reshape(-1)
    theta_flat = student_ability_w.T.reshape(-1)
    h_phys = h_prev.reshape(128, 128, 2).transpose(0, 2, 1).reshape(-1)
    hn_flat, p_correct = _run(
        h_phys, observation, pT_logit, pF_logit, pG_logit, pS_logit,
        om_main, sg_main, tails, theta_flat, kc, pid, sid)
    h_new = hn_flat.reshape(128, 2, 128).transpose(0, 2, 1).reshape(BATCH, 2)
    return (h_new, p_correct)
